# baseline (device time: 304412 ns/iter reference)
import functools

import jax
import jax.numpy as jnp
from jax import lax
from jax.experimental import pallas as pl
from jax.experimental.pallas import tpu as pltpu

N_DEV = 4
SQ = 2048
SKV = 2048
D_MODEL = 1024
HQ_PER = 8
DH = 128
DQ_PER = HQ_PER * DH
HKV = 32 * DH
SCALE = 0.08838834764831843
NEG = -30000.0
QT = 256
GW = 128
WIN = QT + 256
KW = GW + WIN
GR = 32

ORDER = (0, 1, 3, 2)


def _softmax_ctx(s_biased, v):
    e = jnp.exp(s_biased)
    denom = jnp.sum(e, axis=1, keepdims=True)
    w = (e * (1.0 / denom)).astype(jnp.bfloat16)
    return lax.dot_general(
        w, v, (((1,), (0,)), ((), ())), preferred_element_type=jnp.float32
    ).astype(jnp.bfloat16)


def _win_start(r0):
    s0 = jnp.maximum(0, jnp.minimum(r0 - 128, SKV - WIN))
    return pl.multiple_of(s0, 128)


def kernel(x, Wq, K_ext, V_ext, Wo):
    xb = x[0].astype(jnp.bfloat16)
    wq = (Wq * SCALE).astype(jnp.bfloat16)
    wo = Wo.astype(jnp.bfloat16)
    my = lax.axis_index("i")
    k_my = (
        lax.dynamic_index_in_dim(K_ext, my, 0, keepdims=False)
        .astype(jnp.bfloat16)
        .reshape(SKV, HKV)
    )
    v_my = (
        lax.dynamic_index_in_dim(V_ext, my, 0, keepdims=False)
        .astype(jnp.bfloat16)
        .reshape(SKV, HKV)
    )

    def body(
        x_ref,
        wq_ref,
        k_hbm,
        v_hbm,
        wo_ref,
        out_ref,
        wq_buf,
        wo_buf,
        k_scr,
        v_scr,
        bias_scr,
        sendq_sems,
        sendo_sems,
        recvq_sems,
        recvo_sems,
        k_sems,
        v_sems,
    ):
        my_i = lax.axis_index("i")

        barrier_sem = pltpu.get_barrier_semaphore()
        for g in (1, 2, 3):
            pl.semaphore_signal(
                barrier_sem,
                inc=1,
                device_id=((my_i + g) % N_DEV,),
                device_id_type=pl.DeviceIdType.MESH,
            )
        pl.semaphore_wait(barrier_sem, 3)

        wq_buf[my_i] = wq_ref[...]
        wo_buf[my_i] = wo_ref[...]
        sends = []
        for g in (1, 2, 3):
            for buf, ssems, rsems in (
                (wq_buf, sendq_sems, recvq_sems),
                (wo_buf, sendo_sems, recvo_sems),
            ):
                rdma = pltpu.make_async_remote_copy(
                    src_ref=buf.at[my_i],
                    dst_ref=buf.at[my_i],
                    send_sem=ssems.at[g],
                    recv_sem=rsems.at[4 - g],
                    device_id=((my_i + g) % N_DEV,),
                    device_id_type=pl.DeviceIdType.MESH,
                )
                rdma.start()
                sends.append(rdma)

        def kv_dma(j, slot):
            og = (my_i + ORDER[j]) % N_DEV
            cols = pl.ds(og * DQ_PER, DQ_PER)
            kop = pltpu.make_async_copy(
                k_hbm.at[:, cols], k_scr.at[slot], k_sems.at[slot]
            )
            vop = pltpu.make_async_copy(
                v_hbm.at[:, cols], v_scr.at[slot], v_sems.at[slot]
            )
            return kop, vop

        k0, v0 = kv_dma(0, 0)
        k0.start()
        v0.start()

        for qt in range(SQ // QT):
            r0 = qt * QT
            s0 = max(0, min(r0 - 128, SKV - WIN))
            row = lax.broadcasted_iota(jnp.int32, (QT, KW), 0) + r0
            col = lax.broadcasted_iota(jnp.int32, (QT, KW), 1)
            kiw = s0 + col - GW
            win_keep = (col >= GW) & (
                (jnp.abs(row - kiw) <= 128) | (kiw < 32) | (row < 32)
            )
            if s0 >= GW:
                keep = win_keep | (col < 32)
            else:
                keep = win_keep
            bias_scr[r0 : r0 + QT, :] = jnp.where(keep, 0.0, NEG).astype(
                jnp.bfloat16
            )

        for j in range(N_DEV):
            d = ORDER[j]
            o = (my_i + d) % N_DEV
            slot = j % 2

            if j < N_DEV - 1:
                kn, vn = kv_dma(j + 1, 1 - slot)
                kn.start()
                vn.start()

            if d != 0:
                for buf, rsems in ((wq_buf, recvq_sems), (wo_buf, recvo_sems)):
                    recv = pltpu.make_async_remote_copy(
                        src_ref=buf.at[o],
                        dst_ref=buf.at[o],
                        send_sem=sendq_sems.at[0],
                        recv_sem=rsems.at[d],
                        device_id=(my_i,),
                        device_id_type=pl.DeviceIdType.MESH,
                    )
                    recv.wait_recv()
            kw_, vw_ = kv_dma(j, slot)
            kw_.wait()
            vw_.wait()

            q_d = lax.dot_general(
                x_ref[0:GR, :],
                wq_buf[o],
                (((1,), (0,)), ((), ())),
                preferred_element_type=jnp.float32,
            ).astype(jnp.bfloat16)
            d_parts = []
            for h in range(HQ_PER):
                hs = h * DH
                s = lax.dot_general(
                    q_d[:, hs : hs + DH],
                    k_scr[slot, :, hs : hs + DH],
                    (((1,), (1,)), ((), ())),
                    preferred_element_type=jnp.float32,
                )
                d_parts.append(_softmax_ctx(s, v_scr[slot, :, hs : hs + DH]))
            ctx_d = jnp.concatenate(d_parts, axis=1)
            part_d = lax.dot_general(
                ctx_d,
                wo_buf[o],
                (((1,), (0,)), ((), ())),
                preferred_element_type=jnp.float32,
            )
            if j == 0:
                out_ref[0, 0:GR, :] = part_d
            else:
                out_ref[0, 0:GR, :] += part_d

            def tile_step(qt, _):
                r0 = qt * QT
                s0 = _win_start(r0)
                q_tt = lax.dot_general(
                    x_ref[pl.ds(r0, QT), :],
                    wq_buf[o],
                    (((1,), (0,)), ((), ())),
                    preferred_element_type=jnp.float32,
                ).astype(jnp.bfloat16)
                bias_t = bias_scr[pl.ds(r0, QT), :]
                kcat = jnp.concatenate(
                    [k_scr[slot, 0:GW, :], k_scr[slot, pl.ds(s0, WIN), :]],
                    axis=0,
                )
                vcat = jnp.concatenate(
                    [v_scr[slot, 0:GW, :], v_scr[slot, pl.ds(s0, WIN), :]],
                    axis=0,
                )
                parts = []
                for h in range(HQ_PER):
                    hs = h * DH
                    s = lax.dot_general(
                        q_tt[:, hs : hs + DH],
                        kcat[:, hs : hs + DH],
                        (((1,), (1,)), ((), ())),
                        preferred_element_type=jnp.float32,
                    )
                    parts.append(_softmax_ctx(s + bias_t, vcat[:, hs : hs + DH]))
                ctx_t = jnp.concatenate(parts, axis=1)
                part_t = lax.dot_general(
                    ctx_t,
                    wo_buf[o],
                    (((1,), (0,)), ((), ())),
                    preferred_element_type=jnp.float32,
                )

                @pl.when(qt == 0)
                def _():
                    if j == 0:
                        out_ref[0, GR:QT, :] = part_t[GR:QT, :]
                    else:
                        out_ref[0, GR:QT, :] += part_t[GR:QT, :]

                @pl.when(qt > 0)
                def _():
                    if j == 0:
                        out_ref[0, pl.ds(r0, QT), :] = part_t
                    else:
                        out_ref[0, pl.ds(r0, QT), :] += part_t

                return 0

            lax.fori_loop(0, SQ // QT, tile_step, 0)

        for s_ in sends:
            s_.wait_send()

        @functools.partial(pl.run_scoped, sem=pltpu.SemaphoreType.REGULAR)
        def _(sem):
            for g in (1, 2, 3):
                pl.semaphore_signal(
                    sem,
                    inc=1,
                    device_id=((my_i + g) % N_DEV,),
                    device_id_type=pl.DeviceIdType.MESH,
                )
            pl.semaphore_wait(sem, 3)

    return pl.pallas_call(
        body,
        out_shape=jax.ShapeDtypeStruct((1, SQ, D_MODEL), jnp.float32),
        in_specs=[
            pl.BlockSpec(memory_space=pltpu.MemorySpace.VMEM),
            pl.BlockSpec(memory_space=pltpu.MemorySpace.VMEM),
            pl.BlockSpec(memory_space=pl.ANY),
            pl.BlockSpec(memory_space=pl.ANY),
            pl.BlockSpec(memory_space=pltpu.MemorySpace.VMEM),
        ],
        out_specs=pl.BlockSpec(memory_space=pltpu.MemorySpace.VMEM),
        scratch_shapes=[
            pltpu.VMEM((N_DEV, D_MODEL, DQ_PER), jnp.bfloat16),
            pltpu.VMEM((N_DEV, DQ_PER, D_MODEL), jnp.bfloat16),
            pltpu.VMEM((2, SKV, DQ_PER), jnp.bfloat16),
            pltpu.VMEM((2, SKV, DQ_PER), jnp.bfloat16),
            pltpu.VMEM((SQ, KW), jnp.bfloat16),
            pltpu.SemaphoreType.DMA((N_DEV,)),
            pltpu.SemaphoreType.DMA((N_DEV,)),
            pltpu.SemaphoreType.DMA((N_DEV,)),
            pltpu.SemaphoreType.DMA((N_DEV,)),
            pltpu.SemaphoreType.DMA((2,)),
            pltpu.SemaphoreType.DMA((2,)),
        ],
        compiler_params=pltpu.CompilerParams(
            collective_id=0,
            vmem_limit_bytes=110 * 1024 * 1024,
        ),
    )(xb, wq, k_my, v_my, wo)
